# Initial kernel scaffold; baseline (speedup 1.0000x reference)
import jax
import jax.numpy as jnp
from jax.experimental import pallas as pl


def kernel(points, params):
    raise NotImplementedError("write your pallas kernel here")


# speedup vs baseline: 7.4770x; 7.4770x over previous
import functools

import jax
import jax.numpy as jnp
from jax.experimental import pallas as pl
from jax.experimental.pallas import tpu as pltpu

_BIG = 3.0e38
_IBIG = 1 << 30


def _minloc(D, iota):
    m = jnp.min(D, axis=1, keepdims=True)
    am = jnp.min(jnp.where(D == m, iota, _IBIG), axis=1, keepdims=True)
    return m, am


def _sqdist(qc, rc):
    d0 = (qc[0] - rc[0]) ** 2
    d1 = (qc[1] - rc[1]) ** 2
    d2 = (qc[2] - rc[2]) ** 2
    return (d0 + d1) + d2


def _cols(xyz):
    return [xyz[:, c : c + 1] for c in range(3)]


def _rows(xyzT):
    return [xyzT[c : c + 1, :] for c in range(3)]


def _dot(x, w):
    return jnp.dot(x, w, preferred_element_type=jnp.float32)


def _gather_dot(oh, c):
    return jnp.dot(
        oh, c, preferred_element_type=jnp.float32, precision=jax.lax.Precision.HIGHEST
    )


def _lin(x, w_ref, b_ref):
    return _dot(x, w_ref[...]) + b_ref[...]


def _ptb_body(N, d, k, QB, lin0, points_ref, xyzT_ref, *refs):
    if lin0:
        w_l0, b_l0 = refs[0], refs[1]
        refs = refs[2:]
    (w_in, b_in, w_q, b_q, w_k, b_k, w_v, b_v, w_p1, b_p1, w_p2, b_p2,
     w_a1, b_a1, w_a2, b_a2, w_o, b_o) = refs[:18]
    out_ref = refs[18]
    D_scr = refs[19]

    pts = points_ref[0]
    xyz = pts[:, :3]
    if lin0:
        feats = _lin(pts, w_l0, b_l0)
    else:
        feats = pts[:, 3:]
    x = _lin(feats, w_in, b_in)

    kk = _lin(x, w_k, b_k)
    v = _lin(x, w_v, b_v)

    if QB != N:
        qs = pl.program_id(1) * QB
        pts_q = points_ref[0, pl.ds(qs, QB), :]
        qxyz = pts_q[:, :3]
        fq = _lin(pts_q, w_l0, b_l0) if lin0 else pts_q[:, 3:]
        xq = _lin(fq, w_in, b_in)
    else:
        xq, qxyz, fq = x, xyz, feats
    q = _lin(xq, w_q, b_q)
    qc = _cols(qxyz)
    rc = _rows(xyzT_ref[0])

    D_scr[...] = _sqdist(qc, rc)
    iota = jax.lax.broadcasted_iota(jnp.int32, (QB, N), 1)

    C = jnp.concatenate([kk, v, xyz], axis=1)

    def knn_body(_, st):
        M, Z, Y = st
        D = D_scr[...]
        _, am = _minloc(D, iota)
        oh_mask = iota == am
        D_scr[...] = jnp.where(oh_mask, _BIG, D)
        oh = oh_mask.astype(jnp.float32)
        G = _gather_dot(oh, C)
        kj = G[:, :d]
        vj = G[:, d : 2 * d]
        rel = qxyz - G[:, 2 * d :]
        pos = _lin(jnp.maximum(_lin(rel, w_p1, b_p1), 0.0), w_p2, b_p2)
        e = _lin(jnp.maximum(_lin(q - kj + pos, w_a1, b_a1), 0.0), w_a2, b_a2)
        wgt = vj + pos
        Mn = jnp.maximum(M, e)
        scale = jnp.exp(M - Mn)
        s = jnp.exp(e - Mn)
        return Mn, Z * scale + s, Y * scale + s * wgt

    M0 = jnp.full((QB, d), -_BIG, jnp.float32)
    Z0 = jnp.zeros((QB, d), jnp.float32)
    Y0 = jnp.zeros((QB, d), jnp.float32)
    _, Z, Y = jax.lax.fori_loop(0, k, knn_body, (M0, Z0, Y0))
    y = Y / Z
    y = _dot(y, w_o[...]) + b_o[...]
    out_ref[0] = fq + y


def _ptb_call(points, xyzT, p, N, d, k, QB, lin0_wb=None):
    B = points.shape[0]
    in_dim = points.shape[-1]
    nqb = N // QB
    lin0 = lin0_wb is not None
    wb = []
    specs = [
        pl.BlockSpec((1, N, in_dim), lambda b, qb: (b, 0, 0)),
        pl.BlockSpec((1, 3, N), lambda b, qb: (b, 0, 0)),
    ]

    def add_wb(w, bias):
        wb.append(w)
        wb.append(bias.reshape(1, -1))
        specs.append(pl.BlockSpec(w.shape, lambda b, qb: (0, 0)))
        specs.append(pl.BlockSpec((1, bias.shape[0]), lambda b, qb: (0, 0)))

    if lin0:
        add_wb(*lin0_wb)
    for name in ("in", "q", "k", "v", "p1", "p2", "a1", "a2", "out"):
        add_wb(*p[name])
    return pl.pallas_call(
        functools.partial(_ptb_body, N, d, k, QB, lin0),
        grid=(B, nqb),
        in_specs=specs,
        out_specs=pl.BlockSpec((1, QB, d), lambda b, qb: (b, qb, 0)),
        out_shape=jax.ShapeDtypeStruct((B, N, d), jnp.float32),
        scratch_shapes=[pltpu.VMEM((QB, N), jnp.float32)],
        compiler_params=pltpu.CompilerParams(
            dimension_semantics=("parallel", "arbitrary"),
        ),
    )(points, xyzT, *wb)


def _fps_body(B, N, npoint, xyzT_ref, sel_ref):
    xs = xyzT_ref[:, 0, :]
    ys = xyzT_ref[:, 1, :]
    zs = xyzT_ref[:, 2, :]
    iota = jax.lax.broadcasted_iota(jnp.int32, (B, N), 1)

    def coord_at(coords, am):
        return jnp.sum(jnp.where(iota == am, coords, 0.0), axis=1, keepdims=True)

    lx, ly, lz = xs[:, :1], ys[:, :1], zs[:, :1]

    def body(i, st):
        sel, dd, lx, ly, lz = st
        dcur = (xs - lx) ** 2 + (ys - ly) ** 2 + (zs - lz) ** 2
        dd = jnp.minimum(dd, dcur)
        mx = jnp.max(dd, axis=1, keepdims=True)
        am = jnp.min(jnp.where(dd == mx, iota, _IBIG), axis=1, keepdims=True)
        sel = jnp.where(
            jax.lax.broadcasted_iota(jnp.int32, (B, npoint), 1) == i, am, sel
        )
        return sel, dd, coord_at(xs, am), coord_at(ys, am), coord_at(zs, am)

    sel0 = jnp.zeros((B, npoint), jnp.int32)
    dd0 = jnp.full((B, N), 1e10, jnp.float32)
    sel, _, _, _, _ = jax.lax.fori_loop(1, npoint, body, (sel0, dd0, lx, ly, lz))
    sel_ref[...] = sel


def _fps_call(xyzT, npoint):
    B, _, N = xyzT.shape
    return pl.pallas_call(
        functools.partial(_fps_body, B, N, npoint),
        out_shape=jax.ShapeDtypeStruct((B, npoint), jnp.int32),
    )(xyzT)


def _tdb_body(N, npoint, din, dout, k, feats_ref, xyz_ref, xyzT_ref, sel_ref,
              w_ref, b_ref, newxyz_ref, out_ref, D_scr):
    feats = feats_ref[0]
    xyz = xyz_ref[0]
    rc = _rows(xyzT_ref[0])
    sel = sel_ref[0]

    iota_s = jax.lax.broadcasted_iota(jnp.int32, (npoint, N), 1)
    oh_sel = (iota_s == sel).astype(jnp.float32)
    new_xyz = _gather_dot(oh_sel, xyz)

    qc = _cols(new_xyz)
    D_scr[...] = _sqdist(qc, rc)

    C = jnp.concatenate([feats, xyz], axis=1)
    w = w_ref[...]
    bias = b_ref[...]

    def knn_body(_, acc):
        D = D_scr[...]
        _, am = _minloc(D, iota_s)
        oh_mask = iota_s == am
        D_scr[...] = jnp.where(oh_mask, _BIG, D)
        oh = oh_mask.astype(jnp.float32)
        G = _gather_dot(oh, C)
        fj = G[:, :din]
        rel = G[:, din:] - new_xyz
        h = jnp.concatenate([fj, rel], axis=1)
        h = jnp.maximum(_dot(h, w) + bias, 0.0)
        return jnp.maximum(acc, h)

    acc = jax.lax.fori_loop(
        0, k, knn_body, jnp.zeros((npoint, dout), jnp.float32)
    )
    newxyz_ref[0] = new_xyz
    out_ref[0] = acc


def _tdb_call(feats, xyz, xyzT, sel3, p, N, npoint, din, dout, k):
    B = feats.shape[0]
    w, bias = p["mlp"]
    new_xyz, out = pl.pallas_call(
        functools.partial(_tdb_body, N, npoint, din, dout, k),
        grid=(B,),
        in_specs=[
            pl.BlockSpec((1, N, din), lambda b: (b, 0, 0)),
            pl.BlockSpec((1, N, 3), lambda b: (b, 0, 0)),
            pl.BlockSpec((1, 3, N), lambda b: (b, 0, 0)),
            pl.BlockSpec((1, npoint, 1), lambda b: (b, 0, 0)),
            pl.BlockSpec(w.shape, lambda b: (0, 0)),
            pl.BlockSpec((1, dout), lambda b: (0, 0)),
        ],
        out_specs=[
            pl.BlockSpec((1, npoint, 3), lambda b: (b, 0, 0)),
            pl.BlockSpec((1, npoint, dout), lambda b: (b, 0, 0)),
        ],
        out_shape=[
            jax.ShapeDtypeStruct((B, npoint, 3), jnp.float32),
            jax.ShapeDtypeStruct((B, npoint, dout), jnp.float32),
        ],
        scratch_shapes=[pltpu.VMEM((npoint, N), jnp.float32)],
        compiler_params=pltpu.CompilerParams(
            dimension_semantics=("parallel",),
        ),
    )(feats, xyz, xyzT, sel3, w, bias.reshape(1, -1))
    return new_xyz, out


def _head_body(npts, x_ref, w_ref, b_ref, out_ref):
    acc = x_ref[:, 0, :]
    for i in range(1, npts):
        acc = acc + x_ref[:, i, :]
    m = acc / float(npts)
    out_ref[...] = _dot(m, w_ref[...]) + b_ref[...]


def _head_call(x, p):
    B, npts, d = x.shape
    w, bias = p
    nout = w.shape[1]
    return pl.pallas_call(
        functools.partial(_head_body, npts),
        out_shape=jax.ShapeDtypeStruct((B, nout), jnp.float32),
    )(x, w, bias.reshape(1, -1))


def _xyzT_of(xyz):
    return jnp.transpose(xyz, (0, 2, 1))


def kernel(points, params):
    B, N0, _ = points.shape
    xyz = points[:, :, :3]
    xyzT = _xyzT_of(xyz)

    x = _ptb_call(points, xyzT, params["ptb0"], N=N0, d=32, k=16, QB=512,
                  lin0_wb=params["lin0"])

    def tdb_stage(xyz, xyzT, x, p, N, npoint, din, dout, k):
        sel = _fps_call(xyzT, npoint)
        sel3 = sel.reshape(B, npoint, 1)
        new_xyz, out = _tdb_call(x, xyz, xyzT, sel3, p, N, npoint, din, dout, k)
        return new_xyz, _xyzT_of(new_xyz), out

    def ptb_stage(xyz, xyzT, x, p, N, d, k, QB):
        pts = jnp.concatenate([xyz, x], axis=-1)
        return _ptb_call(pts, xyzT, p, N=N, d=d, k=k, QB=QB)

    xyz, xyzT, x = tdb_stage(xyz, xyzT, x, params["tdb1"], 2048, 512, 32, 64, 16)
    x = ptb_stage(xyz, xyzT, x, params["ptb1"], 512, 64, 16, 512)
    xyz, xyzT, x = tdb_stage(xyz, xyzT, x, params["tdb2"], 512, 128, 64, 128, 8)
    x = ptb_stage(xyz, xyzT, x, params["ptb2"], 128, 128, 8, 128)
    xyz, xyzT, x = tdb_stage(xyz, xyzT, x, params["tdb3"], 128, 32, 128, 256, 4)
    x = ptb_stage(xyz, xyzT, x, params["ptb3"], 32, 256, 4, 32)
    xyz, xyzT, x = tdb_stage(xyz, xyzT, x, params["tdb4"], 32, 8, 256, 512, 2)
    x = ptb_stage(xyz, xyzT, x, params["ptb4"], 8, 512, 2, 8)

    return _head_call(x, params["head"])


# speedup vs baseline: 11.7290x; 1.5687x over previous
import functools

import jax
import jax.numpy as jnp
from jax.experimental import pallas as pl
from jax.experimental.pallas import tpu as pltpu

_BIG = 3.0e38
_IBIG = 1 << 30


def _minloc(D, iota):
    m = jnp.min(D, axis=1, keepdims=True)
    am = jnp.min(jnp.where(D == m, iota, _IBIG), axis=1, keepdims=True)
    return m, am


def _sqdist(qc, rc):
    d0 = (qc[0] - rc[0]) ** 2
    d1 = (qc[1] - rc[1]) ** 2
    d2 = (qc[2] - rc[2]) ** 2
    return (d0 + d1) + d2


def _cols(xyz):
    return [xyz[:, c : c + 1] for c in range(3)]


def _rows(xyzT):
    return [xyzT[c : c + 1, :] for c in range(3)]


def _dot(x, w):
    return jnp.dot(x, w, preferred_element_type=jnp.float32)


def _gather_dot_exact(oh, c):
    return jnp.dot(
        oh, c, preferred_element_type=jnp.float32, precision=jax.lax.Precision.HIGHEST
    )


def _split_hi_lo(c):
    hi = c.astype(jnp.bfloat16)
    lo = (c - hi.astype(jnp.float32)).astype(jnp.bfloat16)
    return hi, lo


def _gather_dot2(oh_bf, c_hi, c_lo):
    g_hi = jnp.dot(oh_bf, c_hi, preferred_element_type=jnp.float32)
    g_lo = jnp.dot(oh_bf, c_lo, preferred_element_type=jnp.float32)
    return g_hi + g_lo


def _lin(x, w_ref, b_ref):
    return _dot(x, w_ref[...]) + b_ref[...]


def _ptb_body(N, d, k, QB, lin0, points_ref, xyzT_ref, *refs):
    if lin0:
        w_l0, b_l0 = refs[0], refs[1]
        refs = refs[2:]
    (w_in, b_in, w_q, b_q, w_k, b_k, w_v, b_v, w_p1, b_p1, w_p2, b_p2,
     w_a1, b_a1, w_a2, b_a2, w_o, b_o) = refs[:18]
    out_ref = refs[18]
    D_scr = refs[19]

    pts = points_ref[0]
    xyz = pts[:, :3]
    if lin0:
        feats = _lin(pts, w_l0, b_l0)
    else:
        feats = pts[:, 3:]
    x = _lin(feats, w_in, b_in)

    kk = _lin(x, w_k, b_k)
    v = _lin(x, w_v, b_v)

    if QB != N:
        qs = pl.program_id(1) * QB
        pts_q = points_ref[0, pl.ds(qs, QB), :]
        qxyz = pts_q[:, :3]
        fq = _lin(pts_q, w_l0, b_l0) if lin0 else pts_q[:, 3:]
        xq = _lin(fq, w_in, b_in)
    else:
        xq, qxyz, fq = x, xyz, feats
    q = _lin(xq, w_q, b_q)
    qc = _cols(qxyz)
    rc = _rows(xyzT_ref[0])

    D_scr[...] = _sqdist(qc, rc)
    iota = jax.lax.broadcasted_iota(jnp.int32, (QB, N), 1)

    C = jnp.concatenate([kk, v, xyz], axis=1)
    C_hi, C_lo = _split_hi_lo(C)

    def knn_body(_, st):
        M, Z, Y = st
        D = D_scr[...]
        _, am = _minloc(D, iota)
        oh_mask = iota == am
        D_scr[...] = jnp.where(oh_mask, _BIG, D)
        G = _gather_dot2(oh_mask.astype(jnp.bfloat16), C_hi, C_lo)
        kj = G[:, :d]
        vj = G[:, d : 2 * d]
        rel = qxyz - G[:, 2 * d :]
        pos = _lin(jnp.maximum(_lin(rel, w_p1, b_p1), 0.0), w_p2, b_p2)
        e = _lin(jnp.maximum(_lin(q - kj + pos, w_a1, b_a1), 0.0), w_a2, b_a2)
        wgt = vj + pos
        Mn = jnp.maximum(M, e)
        scale = jnp.exp(M - Mn)
        s = jnp.exp(e - Mn)
        return Mn, Z * scale + s, Y * scale + s * wgt

    M0 = jnp.full((QB, d), -_BIG, jnp.float32)
    Z0 = jnp.zeros((QB, d), jnp.float32)
    Y0 = jnp.zeros((QB, d), jnp.float32)
    _, Z, Y = jax.lax.fori_loop(0, k, knn_body, (M0, Z0, Y0))
    y = Y / Z
    y = _dot(y, w_o[...]) + b_o[...]
    out_ref[0] = fq + y


def _ptb_call(points, xyzT, p, N, d, k, QB, lin0_wb=None):
    B = points.shape[0]
    in_dim = points.shape[-1]
    nqb = N // QB
    lin0 = lin0_wb is not None
    wb = []
    specs = [
        pl.BlockSpec((1, N, in_dim), lambda b, qb: (b, 0, 0)),
        pl.BlockSpec((1, 3, N), lambda b, qb: (b, 0, 0)),
    ]

    def add_wb(w, bias):
        wb.append(w)
        wb.append(bias.reshape(1, -1))
        specs.append(pl.BlockSpec(w.shape, lambda b, qb: (0, 0)))
        specs.append(pl.BlockSpec((1, bias.shape[0]), lambda b, qb: (0, 0)))

    if lin0:
        add_wb(*lin0_wb)
    for name in ("in", "q", "k", "v", "p1", "p2", "a1", "a2", "out"):
        add_wb(*p[name])
    return pl.pallas_call(
        functools.partial(_ptb_body, N, d, k, QB, lin0),
        grid=(B, nqb),
        in_specs=specs,
        out_specs=pl.BlockSpec((1, QB, d), lambda b, qb: (b, qb, 0)),
        out_shape=jax.ShapeDtypeStruct((B, N, d), jnp.float32),
        scratch_shapes=[pltpu.VMEM((QB, N), jnp.float32)],
        compiler_params=pltpu.CompilerParams(
            dimension_semantics=("parallel", "arbitrary"),
        ),
    )(points, xyzT, *wb)


def _fps_body(B, N, npoint, xyzT_ref, sel_ref):
    xs = xyzT_ref[:, 0, :]
    ys = xyzT_ref[:, 1, :]
    zs = xyzT_ref[:, 2, :]
    iota = jax.lax.broadcasted_iota(jnp.int32, (B, N), 1)

    def coord_at(coords, am):
        return jnp.sum(jnp.where(iota == am, coords, 0.0), axis=1, keepdims=True)

    lx, ly, lz = xs[:, :1], ys[:, :1], zs[:, :1]

    def body(i, st):
        sel, dd, lx, ly, lz = st
        dcur = (xs - lx) ** 2 + (ys - ly) ** 2 + (zs - lz) ** 2
        dd = jnp.minimum(dd, dcur)
        mx = jnp.max(dd, axis=1, keepdims=True)
        am = jnp.min(jnp.where(dd == mx, iota, _IBIG), axis=1, keepdims=True)
        sel = jnp.where(
            jax.lax.broadcasted_iota(jnp.int32, (B, npoint), 1) == i, am, sel
        )
        return sel, dd, coord_at(xs, am), coord_at(ys, am), coord_at(zs, am)

    sel0 = jnp.zeros((B, npoint), jnp.int32)
    dd0 = jnp.full((B, N), 1e10, jnp.float32)
    sel, _, _, _, _ = jax.lax.fori_loop(1, npoint, body, (sel0, dd0, lx, ly, lz))
    sel_ref[...] = sel


def _fps_call(xyzT, npoint):
    B, _, N = xyzT.shape
    return pl.pallas_call(
        functools.partial(_fps_body, B, N, npoint),
        out_shape=jax.ShapeDtypeStruct((B, npoint), jnp.int32),
    )(xyzT)


def _tdb_body(N, npoint, din, dout, k, feats_ref, xyz_ref, xyzT_ref, sel_ref,
              w_ref, b_ref, newxyz_ref, out_ref, D_scr):
    feats = feats_ref[0]
    xyz = xyz_ref[0]
    rc = _rows(xyzT_ref[0])
    sel = sel_ref[0]

    iota_s = jax.lax.broadcasted_iota(jnp.int32, (npoint, N), 1)
    oh_sel = (iota_s == sel).astype(jnp.float32)
    new_xyz = _gather_dot_exact(oh_sel, xyz)

    qc = _cols(new_xyz)
    D_scr[...] = _sqdist(qc, rc)

    C = jnp.concatenate([feats, xyz], axis=1)
    C_hi, C_lo = _split_hi_lo(C)
    w = w_ref[...]
    bias = b_ref[...]

    def knn_body(_, acc):
        D = D_scr[...]
        _, am = _minloc(D, iota_s)
        oh_mask = iota_s == am
        D_scr[...] = jnp.where(oh_mask, _BIG, D)
        G = _gather_dot2(oh_mask.astype(jnp.bfloat16), C_hi, C_lo)
        fj = G[:, :din]
        rel = G[:, din:] - new_xyz
        h = jnp.concatenate([fj, rel], axis=1)
        h = jnp.maximum(_dot(h, w) + bias, 0.0)
        return jnp.maximum(acc, h)

    acc = jax.lax.fori_loop(
        0, k, knn_body, jnp.zeros((npoint, dout), jnp.float32)
    )
    newxyz_ref[0] = new_xyz
    out_ref[0] = acc


def _tdb_call(feats, xyz, xyzT, sel3, p, N, npoint, din, dout, k):
    B = feats.shape[0]
    w, bias = p["mlp"]
    new_xyz, out = pl.pallas_call(
        functools.partial(_tdb_body, N, npoint, din, dout, k),
        grid=(B,),
        in_specs=[
            pl.BlockSpec((1, N, din), lambda b: (b, 0, 0)),
            pl.BlockSpec((1, N, 3), lambda b: (b, 0, 0)),
            pl.BlockSpec((1, 3, N), lambda b: (b, 0, 0)),
            pl.BlockSpec((1, npoint, 1), lambda b: (b, 0, 0)),
            pl.BlockSpec(w.shape, lambda b: (0, 0)),
            pl.BlockSpec((1, dout), lambda b: (0, 0)),
        ],
        out_specs=[
            pl.BlockSpec((1, npoint, 3), lambda b: (b, 0, 0)),
            pl.BlockSpec((1, npoint, dout), lambda b: (b, 0, 0)),
        ],
        out_shape=[
            jax.ShapeDtypeStruct((B, npoint, 3), jnp.float32),
            jax.ShapeDtypeStruct((B, npoint, dout), jnp.float32),
        ],
        scratch_shapes=[pltpu.VMEM((npoint, N), jnp.float32)],
        compiler_params=pltpu.CompilerParams(
            dimension_semantics=("parallel",),
        ),
    )(feats, xyz, xyzT, sel3, w, bias.reshape(1, -1))
    return new_xyz, out


def _head_body(npts, x_ref, w_ref, b_ref, out_ref):
    acc = x_ref[:, 0, :]
    for i in range(1, npts):
        acc = acc + x_ref[:, i, :]
    m = acc / float(npts)
    out_ref[...] = _dot(m, w_ref[...]) + b_ref[...]


def _head_call(x, p):
    B, npts, d = x.shape
    w, bias = p
    nout = w.shape[1]
    return pl.pallas_call(
        functools.partial(_head_body, npts),
        out_shape=jax.ShapeDtypeStruct((B, nout), jnp.float32),
    )(x, w, bias.reshape(1, -1))


def _xyzT_of(xyz):
    return jnp.transpose(xyz, (0, 2, 1))


def kernel(points, params):
    B, N0, _ = points.shape
    xyz = points[:, :, :3]
    xyzT = _xyzT_of(xyz)

    x = _ptb_call(points, xyzT, params["ptb0"], N=N0, d=32, k=16, QB=512,
                  lin0_wb=params["lin0"])

    def tdb_stage(xyz, xyzT, x, p, N, npoint, din, dout, k):
        sel = _fps_call(xyzT, npoint)
        sel3 = sel.reshape(B, npoint, 1)
        new_xyz, out = _tdb_call(x, xyz, xyzT, sel3, p, N, npoint, din, dout, k)
        return new_xyz, _xyzT_of(new_xyz), out

    def ptb_stage(xyz, xyzT, x, p, N, d, k, QB):
        pts = jnp.concatenate([xyz, x], axis=-1)
        return _ptb_call(pts, xyzT, p, N=N, d=d, k=k, QB=QB)

    xyz, xyzT, x = tdb_stage(xyz, xyzT, x, params["tdb1"], 2048, 512, 32, 64, 16)
    x = ptb_stage(xyz, xyzT, x, params["ptb1"], 512, 64, 16, 512)
    xyz, xyzT, x = tdb_stage(xyz, xyzT, x, params["tdb2"], 512, 128, 64, 128, 8)
    x = ptb_stage(xyz, xyzT, x, params["ptb2"], 128, 128, 8, 128)
    xyz, xyzT, x = tdb_stage(xyz, xyzT, x, params["tdb3"], 128, 32, 128, 256, 4)
    x = ptb_stage(xyz, xyzT, x, params["ptb3"], 32, 256, 4, 32)
    xyz, xyzT, x = tdb_stage(xyz, xyzT, x, params["tdb4"], 32, 8, 256, 512, 2)
    x = ptb_stage(xyz, xyzT, x, params["ptb4"], 8, 512, 2, 8)

    return _head_call(x, params["head"])


# speedup vs baseline: 11.9270x; 1.0169x over previous
import functools

import jax
import jax.numpy as jnp
from jax.experimental import pallas as pl
from jax.experimental.pallas import tpu as pltpu

_BIG = 3.0e38
_IBIG = 1 << 30


def _minloc(D, iota):
    m = jnp.min(D, axis=1, keepdims=True)
    am = jnp.min(jnp.where(D == m, iota, _IBIG), axis=1, keepdims=True)
    return m, am


def _sqdist(qc, rc):
    d0 = (qc[0] - rc[0]) ** 2
    d1 = (qc[1] - rc[1]) ** 2
    d2 = (qc[2] - rc[2]) ** 2
    return (d0 + d1) + d2


def _cols(xyz):
    return [xyz[:, c : c + 1] for c in range(3)]


def _rep(a, k):
    return jnp.broadcast_to(a[None], (k,) + a.shape).reshape(
        k * a.shape[0], a.shape[1]
    )


def _rows(xyzT):
    return [xyzT[c : c + 1, :] for c in range(3)]


def _dot(x, w):
    return jnp.dot(x, w, preferred_element_type=jnp.float32)


def _gather_dot_exact(oh, c):
    return jnp.dot(
        oh, c, preferred_element_type=jnp.float32, precision=jax.lax.Precision.HIGHEST
    )


def _split_hi_lo(c):
    hi = c.astype(jnp.bfloat16)
    lo = (c - hi.astype(jnp.float32)).astype(jnp.bfloat16)
    return hi, lo


def _gather_dot2(oh_bf, c_hi, c_lo):
    g_hi = jnp.dot(oh_bf, c_hi, preferred_element_type=jnp.float32)
    g_lo = jnp.dot(oh_bf, c_lo, preferred_element_type=jnp.float32)
    return g_hi + g_lo


def _lin(x, w_ref, b_ref):
    return _dot(x, w_ref[...]) + b_ref[...]


def _ptb_body(N, d, k, QB, lin0, points_ref, xyzT_ref, *refs):
    if lin0:
        w_l0, b_l0 = refs[0], refs[1]
        refs = refs[2:]
    (w_in, b_in, w_q, b_q, w_k, b_k, w_v, b_v, w_p1, b_p1, w_p2, b_p2,
     w_a1, b_a1, w_a2, b_a2, w_o, b_o) = refs[:18]
    out_ref = refs[18]
    D_scr = refs[19]
    OH_scr = refs[20]

    pts = points_ref[0]
    xyz = pts[:, :3]
    if lin0:
        feats = _lin(pts, w_l0, b_l0)
    else:
        feats = pts[:, 3:]
    x = _lin(feats, w_in, b_in)

    kk = _lin(x, w_k, b_k)
    v = _lin(x, w_v, b_v)

    if QB != N:
        qs = pl.program_id(1) * QB
        pts_q = points_ref[0, pl.ds(qs, QB), :]
        qxyz = pts_q[:, :3]
        fq = _lin(pts_q, w_l0, b_l0) if lin0 else pts_q[:, 3:]
        xq = _lin(fq, w_in, b_in)
    else:
        xq, qxyz, fq = x, xyz, feats
    q = _lin(xq, w_q, b_q)
    qc = _cols(qxyz)
    rc = _rows(xyzT_ref[0])

    D_scr[...] = _sqdist(qc, rc)
    iota = jax.lax.broadcasted_iota(jnp.int32, (QB, N), 1)

    C = jnp.concatenate([kk, v, xyz], axis=1)
    C_hi, C_lo = _split_hi_lo(C)

    def topk_body(j, carry):
        D = D_scr[...]
        _, am = _minloc(D, iota)
        oh_mask = iota == am
        D_scr[...] = jnp.where(oh_mask, _BIG, D)
        OH_scr[pl.ds(j * QB, QB), :] = oh_mask.astype(jnp.bfloat16)
        return carry

    jax.lax.fori_loop(0, k, topk_body, 0)

    Gall = _gather_dot2(OH_scr[...], C_hi, C_lo)

    kj = Gall[:, :d]
    vj = Gall[:, d : 2 * d]
    nxyz = Gall[:, 2 * d :]
    q_rep = _rep(q, k)
    rel = _rep(qxyz, k) - nxyz
    pos = _lin(jnp.maximum(_lin(rel, w_p1, b_p1), 0.0), w_p2, b_p2)
    e = _lin(jnp.maximum(_lin(q_rep - kj + pos, w_a1, b_a1), 0.0), w_a2, b_a2)
    wgt = (vj + pos).reshape(k, QB, d)
    e = e.reshape(k, QB, d)
    m = e[0]
    for j in range(1, k):
        m = jnp.maximum(m, e[j])
    Z = Y = None
    for j in range(k):
        s = jnp.exp(e[j] - m)
        Z = s if Z is None else Z + s
        sw = s * wgt[j]
        Y = sw if Y is None else Y + sw
    y = Y / Z
    y = _dot(y, w_o[...]) + b_o[...]
    out_ref[0] = fq + y


def _ptb_call(points, xyzT, p, N, d, k, QB, lin0_wb=None):
    B = points.shape[0]
    in_dim = points.shape[-1]
    nqb = N // QB
    lin0 = lin0_wb is not None
    wb = []
    specs = [
        pl.BlockSpec((1, N, in_dim), lambda b, qb: (b, 0, 0)),
        pl.BlockSpec((1, 3, N), lambda b, qb: (b, 0, 0)),
    ]

    def add_wb(w, bias):
        wb.append(w)
        wb.append(bias.reshape(1, -1))
        specs.append(pl.BlockSpec(w.shape, lambda b, qb: (0, 0)))
        specs.append(pl.BlockSpec((1, bias.shape[0]), lambda b, qb: (0, 0)))

    if lin0:
        add_wb(*lin0_wb)
    for name in ("in", "q", "k", "v", "p1", "p2", "a1", "a2", "out"):
        add_wb(*p[name])
    return pl.pallas_call(
        functools.partial(_ptb_body, N, d, k, QB, lin0),
        grid=(B, nqb),
        in_specs=specs,
        out_specs=pl.BlockSpec((1, QB, d), lambda b, qb: (b, qb, 0)),
        out_shape=jax.ShapeDtypeStruct((B, N, d), jnp.float32),
        scratch_shapes=[
            pltpu.VMEM((QB, N), jnp.float32),
            pltpu.VMEM((k * QB, N), jnp.bfloat16),
        ],
        compiler_params=pltpu.CompilerParams(
            dimension_semantics=("parallel", "arbitrary"),
        ),
    )(points, xyzT, *wb)


def _fps_body(B, N, npoint, xyzT_ref, sel_ref):
    xs = xyzT_ref[:, 0, :]
    ys = xyzT_ref[:, 1, :]
    zs = xyzT_ref[:, 2, :]
    iota = jax.lax.broadcasted_iota(jnp.int32, (B, N), 1)

    def coord_at(coords, am):
        return jnp.sum(jnp.where(iota == am, coords, 0.0), axis=1, keepdims=True)

    lx, ly, lz = xs[:, :1], ys[:, :1], zs[:, :1]

    def body(i, st):
        sel, dd, lx, ly, lz = st
        dcur = (xs - lx) ** 2 + (ys - ly) ** 2 + (zs - lz) ** 2
        dd = jnp.minimum(dd, dcur)
        mx = jnp.max(dd, axis=1, keepdims=True)
        am = jnp.min(jnp.where(dd == mx, iota, _IBIG), axis=1, keepdims=True)
        sel = jnp.where(
            jax.lax.broadcasted_iota(jnp.int32, (B, npoint), 1) == i, am, sel
        )
        return sel, dd, coord_at(xs, am), coord_at(ys, am), coord_at(zs, am)

    sel0 = jnp.zeros((B, npoint), jnp.int32)
    dd0 = jnp.full((B, N), 1e10, jnp.float32)
    sel, _, _, _, _ = jax.lax.fori_loop(1, npoint, body, (sel0, dd0, lx, ly, lz))
    sel_ref[...] = sel


def _fps_call(xyzT, npoint):
    B, _, N = xyzT.shape
    return pl.pallas_call(
        functools.partial(_fps_body, B, N, npoint),
        out_shape=jax.ShapeDtypeStruct((B, npoint), jnp.int32),
    )(xyzT)


def _tdb_body(N, QBT, din, dout, k, feats_ref, xyz_ref, xyzT_ref, sel_ref,
              w_ref, b_ref, newxyz_ref, out_ref, D_scr, OH_scr):
    npoint = QBT
    feats = feats_ref[0]
    xyz = xyz_ref[0]
    rc = _rows(xyzT_ref[0])
    sel = sel_ref[0]

    iota_s = jax.lax.broadcasted_iota(jnp.int32, (npoint, N), 1)
    oh_sel = (iota_s == sel).astype(jnp.float32)
    new_xyz = _gather_dot_exact(oh_sel, xyz)

    qc = _cols(new_xyz)
    D_scr[...] = _sqdist(qc, rc)

    C = jnp.concatenate([feats, xyz], axis=1)
    C_hi, C_lo = _split_hi_lo(C)
    w = w_ref[...]
    bias = b_ref[...]

    def topk_body(j, carry):
        D = D_scr[...]
        _, am = _minloc(D, iota_s)
        oh_mask = iota_s == am
        D_scr[...] = jnp.where(oh_mask, _BIG, D)
        OH_scr[pl.ds(j * npoint, npoint), :] = oh_mask.astype(jnp.bfloat16)
        return carry

    jax.lax.fori_loop(0, k, topk_body, 0)

    Gall = _gather_dot2(OH_scr[...], C_hi, C_lo)
    fj = Gall[:, :din]
    rel = Gall[:, din:] - _rep(new_xyz, k)
    h = jnp.concatenate([fj, rel], axis=1)
    h = jnp.maximum(_dot(h, w) + bias, 0.0).reshape(k, npoint, dout)
    acc = h[0]
    for j in range(1, k):
        acc = jnp.maximum(acc, h[j])
    newxyz_ref[0] = new_xyz
    out_ref[0] = acc


def _tdb_call(feats, xyz, xyzT, sel3, p, N, npoint, din, dout, k, QBT=None):
    B = feats.shape[0]
    QBT = npoint if QBT is None else QBT
    w, bias = p["mlp"]
    new_xyz, out = pl.pallas_call(
        functools.partial(_tdb_body, N, QBT, din, dout, k),
        grid=(B, npoint // QBT),
        in_specs=[
            pl.BlockSpec((1, N, din), lambda b, qb: (b, 0, 0)),
            pl.BlockSpec((1, N, 3), lambda b, qb: (b, 0, 0)),
            pl.BlockSpec((1, 3, N), lambda b, qb: (b, 0, 0)),
            pl.BlockSpec((1, QBT, 1), lambda b, qb: (b, qb, 0)),
            pl.BlockSpec(w.shape, lambda b, qb: (0, 0)),
            pl.BlockSpec((1, dout), lambda b, qb: (0, 0)),
        ],
        out_specs=[
            pl.BlockSpec((1, QBT, 3), lambda b, qb: (b, qb, 0)),
            pl.BlockSpec((1, QBT, dout), lambda b, qb: (b, qb, 0)),
        ],
        out_shape=[
            jax.ShapeDtypeStruct((B, npoint, 3), jnp.float32),
            jax.ShapeDtypeStruct((B, npoint, dout), jnp.float32),
        ],
        scratch_shapes=[
            pltpu.VMEM((QBT, N), jnp.float32),
            pltpu.VMEM((k * QBT, N), jnp.bfloat16),
        ],
        compiler_params=pltpu.CompilerParams(
            dimension_semantics=("parallel", "arbitrary"),
        ),
    )(feats, xyz, xyzT, sel3, w, bias.reshape(1, -1))
    return new_xyz, out


def _head_body(npts, x_ref, w_ref, b_ref, out_ref):
    acc = x_ref[:, 0, :]
    for i in range(1, npts):
        acc = acc + x_ref[:, i, :]
    m = acc / float(npts)
    out_ref[...] = _dot(m, w_ref[...]) + b_ref[...]


def _head_call(x, p):
    B, npts, d = x.shape
    w, bias = p
    nout = w.shape[1]
    return pl.pallas_call(
        functools.partial(_head_body, npts),
        out_shape=jax.ShapeDtypeStruct((B, nout), jnp.float32),
    )(x, w, bias.reshape(1, -1))


def _xyzT_of(xyz):
    return jnp.transpose(xyz, (0, 2, 1))


def kernel(points, params):
    B, N0, _ = points.shape
    xyz = points[:, :, :3]
    xyzT = _xyzT_of(xyz)

    x = _ptb_call(points, xyzT, params["ptb0"], N=N0, d=32, k=16, QB=256,
                  lin0_wb=params["lin0"])

    def tdb_stage(xyz, xyzT, x, p, N, npoint, din, dout, k, QBT=None):
        sel = _fps_call(xyzT, npoint)
        sel3 = sel.reshape(B, npoint, 1)
        new_xyz, out = _tdb_call(x, xyz, xyzT, sel3, p, N, npoint, din, dout, k,
                                 QBT=QBT)
        return new_xyz, _xyzT_of(new_xyz), out

    def ptb_stage(xyz, xyzT, x, p, N, d, k, QB):
        pts = jnp.concatenate([xyz, x], axis=-1)
        return _ptb_call(pts, xyzT, p, N=N, d=d, k=k, QB=QB)

    xyz, xyzT, x = tdb_stage(xyz, xyzT, x, params["tdb1"], 2048, 512, 32, 64, 16,
                             QBT=256)
    x = ptb_stage(xyz, xyzT, x, params["ptb1"], 512, 64, 16, 512)
    xyz, xyzT, x = tdb_stage(xyz, xyzT, x, params["tdb2"], 512, 128, 64, 128, 8)
    x = ptb_stage(xyz, xyzT, x, params["ptb2"], 128, 128, 8, 128)
    xyz, xyzT, x = tdb_stage(xyz, xyzT, x, params["tdb3"], 128, 32, 128, 256, 4)
    x = ptb_stage(xyz, xyzT, x, params["ptb3"], 32, 256, 4, 32)
    xyz, xyzT, x = tdb_stage(xyz, xyzT, x, params["tdb4"], 32, 8, 256, 512, 2)
    x = ptb_stage(xyz, xyzT, x, params["ptb4"], 8, 512, 2, 8)

    return _head_call(x, params["head"])


# speedup vs baseline: 12.0712x; 1.0121x over previous
import functools

import jax
import jax.numpy as jnp
from jax.experimental import pallas as pl
from jax.experimental.pallas import tpu as pltpu

_BIG = 3.0e38
_IBIG = 1 << 30


def _minloc(D, iota):
    m = jnp.min(D, axis=1, keepdims=True)
    am = jnp.min(jnp.where(D == m, iota, _IBIG), axis=1, keepdims=True)
    return m, am


def _sqdist(qc, rc):
    d0 = (qc[0] - rc[0]) ** 2
    d1 = (qc[1] - rc[1]) ** 2
    d2 = (qc[2] - rc[2]) ** 2
    return (d0 + d1) + d2


def _cols(xyz):
    return [xyz[:, c : c + 1] for c in range(3)]


def _rep(a, k):
    return jnp.broadcast_to(a[None], (k,) + a.shape).reshape(
        k * a.shape[0], a.shape[1]
    )


def _rows(xyzT):
    return [xyzT[c : c + 1, :] for c in range(3)]


def _dot(x, w):
    return jnp.dot(x, w, preferred_element_type=jnp.float32)


def _gather_dot_exact(oh, c):
    return jnp.dot(
        oh, c, preferred_element_type=jnp.float32, precision=jax.lax.Precision.HIGHEST
    )


def _split_hi_lo(c):
    hi = c.astype(jnp.bfloat16)
    lo = (c - hi.astype(jnp.float32)).astype(jnp.bfloat16)
    return hi, lo


def _gather_dot2(oh_bf, c_hi, c_lo):
    g_hi = jnp.dot(oh_bf, c_hi, preferred_element_type=jnp.float32)
    g_lo = jnp.dot(oh_bf, c_lo, preferred_element_type=jnp.float32)
    return g_hi + g_lo


def _lin(x, w_ref, b_ref):
    return _dot(x, w_ref[...]) + b_ref[...]


def _ptb_body(N, d, k, QB, lin0, points_ref, xyzT_ref, *refs):
    if lin0:
        w_l0, b_l0 = refs[0], refs[1]
        refs = refs[2:]
    (w_in, b_in, w_q, b_q, w_k, b_k, w_v, b_v, w_p1, b_p1, w_p2, b_p2,
     w_a1, b_a1, w_a2, b_a2, w_o, b_o) = refs[:18]
    out_ref = refs[18]
    D_scr = refs[19]
    OH_scr = refs[20]

    pts = points_ref[0]
    xyz = pts[:, :3]
    if lin0:
        feats = _lin(pts, w_l0, b_l0)
    else:
        feats = pts[:, 3:]
    x = _lin(feats, w_in, b_in)

    kk = _lin(x, w_k, b_k)
    v = _lin(x, w_v, b_v)

    if QB != N:
        qs = pl.program_id(1) * QB
        pts_q = points_ref[0, pl.ds(qs, QB), :]
        qxyz = pts_q[:, :3]
        fq = _lin(pts_q, w_l0, b_l0) if lin0 else pts_q[:, 3:]
        xq = _lin(fq, w_in, b_in)
    else:
        xq, qxyz, fq = x, xyz, feats
    q = _lin(xq, w_q, b_q)
    qc = _cols(qxyz)
    rc = _rows(xyzT_ref[0])

    D_scr[...] = _sqdist(qc, rc)
    iota = jax.lax.broadcasted_iota(jnp.int32, (QB, N), 1)

    C = jnp.concatenate([kk, v, xyz], axis=1)
    C_hi, C_lo = _split_hi_lo(C)

    def topk_body(j, carry):
        D = D_scr[...]
        m = jnp.min(D, axis=1, keepdims=True)
        eq = D == m
        am = jnp.min(jnp.where(eq, iota, _IBIG), axis=1, keepdims=True)
        D_scr[...] = jnp.where(eq, _BIG, D)
        OH_scr[pl.ds(j * QB, QB), :] = (iota == am).astype(jnp.bfloat16)
        return carry

    jax.lax.fori_loop(0, k, topk_body, 0)

    Gall = _gather_dot2(OH_scr[...], C_hi, C_lo)

    kj = Gall[:, :d]
    vj = Gall[:, d : 2 * d]
    nxyz = Gall[:, 2 * d :]
    q_rep = _rep(q, k)
    rel = _rep(qxyz, k) - nxyz
    pos = _lin(jnp.maximum(_lin(rel, w_p1, b_p1), 0.0), w_p2, b_p2)
    e = _lin(jnp.maximum(_lin(q_rep - kj + pos, w_a1, b_a1), 0.0), w_a2, b_a2)
    wgt = (vj + pos).reshape(k, QB, d)
    e = e.reshape(k, QB, d)
    m = e[0]
    for j in range(1, k):
        m = jnp.maximum(m, e[j])
    Z = Y = None
    for j in range(k):
        s = jnp.exp(e[j] - m)
        Z = s if Z is None else Z + s
        sw = s * wgt[j]
        Y = sw if Y is None else Y + sw
    y = Y / Z
    y = _dot(y, w_o[...]) + b_o[...]
    out_ref[0] = fq + y


def _ptb_call(points, xyzT, p, N, d, k, QB, lin0_wb=None):
    B = points.shape[0]
    in_dim = points.shape[-1]
    nqb = N // QB
    lin0 = lin0_wb is not None
    wb = []
    specs = [
        pl.BlockSpec((1, N, in_dim), lambda b, qb: (b, 0, 0)),
        pl.BlockSpec((1, 3, N), lambda b, qb: (b, 0, 0)),
    ]

    def add_wb(w, bias):
        wb.append(w)
        wb.append(bias.reshape(1, -1))
        specs.append(pl.BlockSpec(w.shape, lambda b, qb: (0, 0)))
        specs.append(pl.BlockSpec((1, bias.shape[0]), lambda b, qb: (0, 0)))

    if lin0:
        add_wb(*lin0_wb)
    for name in ("in", "q", "k", "v", "p1", "p2", "a1", "a2", "out"):
        add_wb(*p[name])
    return pl.pallas_call(
        functools.partial(_ptb_body, N, d, k, QB, lin0),
        grid=(B, nqb),
        in_specs=specs,
        out_specs=pl.BlockSpec((1, QB, d), lambda b, qb: (b, qb, 0)),
        out_shape=jax.ShapeDtypeStruct((B, N, d), jnp.float32),
        scratch_shapes=[
            pltpu.VMEM((QB, N), jnp.float32),
            pltpu.VMEM((k * QB, N), jnp.bfloat16),
        ],
        compiler_params=pltpu.CompilerParams(
            dimension_semantics=("parallel", "arbitrary"),
        ),
    )(points, xyzT, *wb)


def _fps_body(B, N, npoint, xyzT_ref, sel_ref):
    xs = xyzT_ref[:, 0, :]
    ys = xyzT_ref[:, 1, :]
    zs = xyzT_ref[:, 2, :]
    iota = jax.lax.broadcasted_iota(jnp.int32, (B, N), 1)

    def coord_at(coords, am):
        return jnp.sum(jnp.where(iota == am, coords, 0.0), axis=1, keepdims=True)

    lx, ly, lz = xs[:, :1], ys[:, :1], zs[:, :1]

    def body(i, st):
        sel, dd, lx, ly, lz = st
        dcur = (xs - lx) ** 2 + (ys - ly) ** 2 + (zs - lz) ** 2
        dd = jnp.minimum(dd, dcur)
        mx = jnp.max(dd, axis=1, keepdims=True)
        am = jnp.min(jnp.where(dd == mx, iota, _IBIG), axis=1, keepdims=True)
        sel = jnp.where(
            jax.lax.broadcasted_iota(jnp.int32, (B, npoint), 1) == i, am, sel
        )
        return sel, dd, coord_at(xs, am), coord_at(ys, am), coord_at(zs, am)

    sel0 = jnp.zeros((B, npoint), jnp.int32)
    dd0 = jnp.full((B, N), 1e10, jnp.float32)
    sel, _, _, _, _ = jax.lax.fori_loop(1, npoint, body, (sel0, dd0, lx, ly, lz))
    sel_ref[...] = sel


def _fps_call(xyzT, npoint):
    B, _, N = xyzT.shape
    return pl.pallas_call(
        functools.partial(_fps_body, B, N, npoint),
        out_shape=jax.ShapeDtypeStruct((B, npoint), jnp.int32),
    )(xyzT)


def _tdb_body(N, QBT, din, dout, k, feats_ref, xyz_ref, xyzT_ref, sel_ref,
              w_ref, b_ref, newxyz_ref, out_ref, D_scr, OH_scr):
    npoint = QBT
    feats = feats_ref[0]
    xyz = xyz_ref[0]
    rc = _rows(xyzT_ref[0])
    sel = sel_ref[0]

    iota_s = jax.lax.broadcasted_iota(jnp.int32, (npoint, N), 1)
    oh_sel = (iota_s == sel).astype(jnp.float32)
    new_xyz = _gather_dot_exact(oh_sel, xyz)

    qc = _cols(new_xyz)
    D_scr[...] = _sqdist(qc, rc)

    C = jnp.concatenate([feats, xyz], axis=1)
    C_hi, C_lo = _split_hi_lo(C)
    w = w_ref[...]
    bias = b_ref[...]

    def topk_body(j, carry):
        D = D_scr[...]
        m = jnp.min(D, axis=1, keepdims=True)
        eq = D == m
        am = jnp.min(jnp.where(eq, iota_s, _IBIG), axis=1, keepdims=True)
        D_scr[...] = jnp.where(eq, _BIG, D)
        OH_scr[pl.ds(j * npoint, npoint), :] = (iota_s == am).astype(jnp.bfloat16)
        return carry

    jax.lax.fori_loop(0, k, topk_body, 0)

    Gall = _gather_dot2(OH_scr[...], C_hi, C_lo)
    fj = Gall[:, :din]
    rel = Gall[:, din:] - _rep(new_xyz, k)
    h = jnp.concatenate([fj, rel], axis=1)
    h = jnp.maximum(_dot(h, w) + bias, 0.0).reshape(k, npoint, dout)
    acc = h[0]
    for j in range(1, k):
        acc = jnp.maximum(acc, h[j])
    newxyz_ref[0] = new_xyz
    out_ref[0] = acc


def _tdb_call(feats, xyz, xyzT, sel3, p, N, npoint, din, dout, k, QBT=None):
    B = feats.shape[0]
    QBT = npoint if QBT is None else QBT
    w, bias = p["mlp"]
    new_xyz, out = pl.pallas_call(
        functools.partial(_tdb_body, N, QBT, din, dout, k),
        grid=(B, npoint // QBT),
        in_specs=[
            pl.BlockSpec((1, N, din), lambda b, qb: (b, 0, 0)),
            pl.BlockSpec((1, N, 3), lambda b, qb: (b, 0, 0)),
            pl.BlockSpec((1, 3, N), lambda b, qb: (b, 0, 0)),
            pl.BlockSpec((1, QBT, 1), lambda b, qb: (b, qb, 0)),
            pl.BlockSpec(w.shape, lambda b, qb: (0, 0)),
            pl.BlockSpec((1, dout), lambda b, qb: (0, 0)),
        ],
        out_specs=[
            pl.BlockSpec((1, QBT, 3), lambda b, qb: (b, qb, 0)),
            pl.BlockSpec((1, QBT, dout), lambda b, qb: (b, qb, 0)),
        ],
        out_shape=[
            jax.ShapeDtypeStruct((B, npoint, 3), jnp.float32),
            jax.ShapeDtypeStruct((B, npoint, dout), jnp.float32),
        ],
        scratch_shapes=[
            pltpu.VMEM((QBT, N), jnp.float32),
            pltpu.VMEM((k * QBT, N), jnp.bfloat16),
        ],
        compiler_params=pltpu.CompilerParams(
            dimension_semantics=("parallel", "arbitrary"),
        ),
    )(feats, xyz, xyzT, sel3, w, bias.reshape(1, -1))
    return new_xyz, out


def _head_body(npts, x_ref, w_ref, b_ref, out_ref):
    acc = x_ref[:, 0, :]
    for i in range(1, npts):
        acc = acc + x_ref[:, i, :]
    m = acc / float(npts)
    out_ref[...] = _dot(m, w_ref[...]) + b_ref[...]


def _head_call(x, p):
    B, npts, d = x.shape
    w, bias = p
    nout = w.shape[1]
    return pl.pallas_call(
        functools.partial(_head_body, npts),
        out_shape=jax.ShapeDtypeStruct((B, nout), jnp.float32),
    )(x, w, bias.reshape(1, -1))


def _xyzT_of(xyz):
    return jnp.transpose(xyz, (0, 2, 1))


def kernel(points, params):
    B, N0, _ = points.shape
    xyz = points[:, :, :3]
    xyzT = _xyzT_of(xyz)

    x = _ptb_call(points, xyzT, params["ptb0"], N=N0, d=32, k=16, QB=256,
                  lin0_wb=params["lin0"])

    def tdb_stage(xyz, xyzT, x, p, N, npoint, din, dout, k, QBT=None):
        sel = _fps_call(xyzT, npoint)
        sel3 = sel.reshape(B, npoint, 1)
        new_xyz, out = _tdb_call(x, xyz, xyzT, sel3, p, N, npoint, din, dout, k,
                                 QBT=QBT)
        return new_xyz, _xyzT_of(new_xyz), out

    def ptb_stage(xyz, xyzT, x, p, N, d, k, QB):
        pts = jnp.concatenate([xyz, x], axis=-1)
        return _ptb_call(pts, xyzT, p, N=N, d=d, k=k, QB=QB)

    xyz, xyzT, x = tdb_stage(xyz, xyzT, x, params["tdb1"], 2048, 512, 32, 64, 16,
                             QBT=256)
    x = ptb_stage(xyz, xyzT, x, params["ptb1"], 512, 64, 16, 512)
    xyz, xyzT, x = tdb_stage(xyz, xyzT, x, params["tdb2"], 512, 128, 64, 128, 8)
    x = ptb_stage(xyz, xyzT, x, params["ptb2"], 128, 128, 8, 128)
    xyz, xyzT, x = tdb_stage(xyz, xyzT, x, params["tdb3"], 128, 32, 128, 256, 4)
    x = ptb_stage(xyz, xyzT, x, params["ptb3"], 32, 256, 4, 32)
    xyz, xyzT, x = tdb_stage(xyz, xyzT, x, params["tdb4"], 32, 8, 256, 512, 2)
    x = ptb_stage(xyz, xyzT, x, params["ptb4"], 8, 512, 2, 8)

    return _head_call(x, params["head"])
